# baseline (speedup 1.0000x reference)
import functools

import jax
import jax.numpy as jnp
from jax import lax
from jax.experimental import pallas as pl
from jax.experimental.pallas import tpu as pltpu
from jax.experimental.pallas import tpu_sc as plsc

NC = 2
NS = 16
NW = NC * NS
NBUF = 8
TBS = 8192


def _transpose_body(x_ref, o_ref):
    d = x_ref.shape[0]
    eye = jnp.eye(d, dtype=jnp.float32)
    o_ref[...] = jax.lax.dot_general(
        x_ref[...], eye, (((0,), (0,)), ((), ())),
        preferred_element_type=jnp.float32,
    )


def _to_rows(w_t):
    d, v = w_t.shape
    return pl.pallas_call(
        _transpose_body,
        grid=(pl.cdiv(v, TBS),),
        in_specs=[pl.BlockSpec((d, TBS), lambda c: (0, c))],
        out_specs=pl.BlockSpec((TBS, d), lambda c: (c, 0)),
        out_shape=jax.ShapeDtypeStruct((v, d), jnp.float32),
    )(w_t)


def kernel(idx, weight):
    b, h = idx.shape
    v, d = weight.shape
    bpw = b // NW
    mesh = plsc.VectorSubcoreMesh(core_axis_name="c", subcore_axis_name="s")

    @functools.partial(
        pl.kernel,
        mesh=mesh,
        compiler_params=pltpu.CompilerParams(use_tc_tiling_on_sc=False),
        out_type=jax.ShapeDtypeStruct((b, h, d), jnp.float32),
        scratch_types=[
            pltpu.VMEM((bpw, h), jnp.int32),
            pltpu.VMEM((NBUF, h, d), jnp.float32),
            pltpu.SemaphoreType.DMA,
            pltpu.SemaphoreType.DMA,
        ],
    )
    def k(idx_hbm, tbl_hbm, out_hbm, idx_v, rows_v, gsem, ssem):
        wid = lax.axis_index("s") * NC + lax.axis_index("c")
        base = wid * bpw
        pltpu.sync_copy(idx_hbm.at[pl.ds(base, bpw)], idx_v)

        def gather(g, buf):
            pltpu.async_copy(tbl_hbm.at[idx_v.at[g]], rows_v.at[buf], gsem)

        for i in range(NBUF):
            gather(i, i)

        def body(j, carry):
            buf = lax.rem(j, NBUF)
            pltpu.make_async_copy(
                tbl_hbm.at[idx_v.at[j]], rows_v.at[buf], gsem
            ).wait()
            pltpu.async_copy(rows_v.at[buf], out_hbm.at[base + j], ssem)

            @pl.when((j >= 1) & (j <= bpw - NBUF))
            def _():
                pltpu.make_async_copy(
                    rows_v.at[buf], out_hbm.at[base], ssem
                ).wait()
                g = j - 1 + NBUF
                gather(g, lax.rem(g, NBUF))

            return carry

        lax.fori_loop(0, bpw, body, 0)

        for _ in range(NBUF):
            pltpu.make_async_copy(
                rows_v.at[0], out_hbm.at[base], ssem
            ).wait()

    return k(idx, _to_rows(weight.T))


# speedup vs baseline: 1.0293x; 1.0293x over previous
import functools

import jax
import jax.numpy as jnp
from jax import lax
from jax.experimental import pallas as pl
from jax.experimental.pallas import tpu as pltpu
from jax.experimental.pallas import tpu_sc as plsc

NC = 2
NS = 16
NW = NC * NS
NBUF = 8
TBS = 16384


def _transpose_body(x_ref, o_ref):
    d = x_ref.shape[0]
    eye = jnp.eye(d, dtype=jnp.float32)
    o_ref[...] = jax.lax.dot_general(
        x_ref[...], eye, (((0,), (0,)), ((), ())),
        preferred_element_type=jnp.float32,
    )


def _to_rows(w_t):
    d, v = w_t.shape
    return pl.pallas_call(
        _transpose_body,
        grid=(pl.cdiv(v, TBS),),
        in_specs=[pl.BlockSpec((d, TBS), lambda c: (0, c))],
        out_specs=pl.BlockSpec((TBS, d), lambda c: (c, 0)),
        out_shape=jax.ShapeDtypeStruct((v, d), jnp.float32),
    )(w_t)


def kernel(idx, weight):
    b, h = idx.shape
    v, d = weight.shape
    bpw = b // NW
    mesh = plsc.VectorSubcoreMesh(core_axis_name="c", subcore_axis_name="s")

    @functools.partial(
        pl.kernel,
        mesh=mesh,
        compiler_params=pltpu.CompilerParams(use_tc_tiling_on_sc=False),
        out_type=jax.ShapeDtypeStruct((b, h, d), jnp.float32),
        scratch_types=[
            pltpu.VMEM((bpw, h), jnp.int32),
            pltpu.VMEM((NBUF, h, d), jnp.float32),
            pltpu.SemaphoreType.DMA,
            pltpu.SemaphoreType.DMA,
        ],
    )
    def k(idx_hbm, tbl_hbm, out_hbm, idx_v, rows_v, gsem, ssem):
        wid = lax.axis_index("s") * NC + lax.axis_index("c")
        base = wid * bpw
        pltpu.sync_copy(idx_hbm.at[pl.ds(base, bpw)], idx_v)

        def gather(g, buf):
            pltpu.async_copy(tbl_hbm.at[idx_v.at[g]], rows_v.at[buf], gsem)

        for i in range(NBUF):
            gather(i, i)

        def body(j, carry):
            buf = lax.rem(j, NBUF)
            pltpu.make_async_copy(
                tbl_hbm.at[idx_v.at[j]], rows_v.at[buf], gsem
            ).wait()
            pltpu.async_copy(rows_v.at[buf], out_hbm.at[base + j], ssem)

            @pl.when((j >= 1) & (j <= bpw - NBUF))
            def _():
                pltpu.make_async_copy(
                    rows_v.at[buf], out_hbm.at[base], ssem
                ).wait()
                g = j - 1 + NBUF
                gather(g, lax.rem(g, NBUF))

            return carry

        lax.fori_loop(0, bpw, body, 0)

        for _ in range(NBUF):
            pltpu.make_async_copy(
                rows_v.at[0], out_hbm.at[base], ssem
            ).wait()

    return k(idx, _to_rows(weight.T))


# speedup vs baseline: 1.0735x; 1.0429x over previous
import functools

import jax
import jax.numpy as jnp
from jax import lax
from jax.experimental import pallas as pl
from jax.experimental.pallas import tpu as pltpu
from jax.experimental.pallas import tpu_sc as plsc

NC = 2
NS = 16
NW = NC * NS
CHUNK = 128
VG = 16
NBUF = 8


def kernel(idx, weight):
    b, h, (v, d) = idx.shape[0], idx.shape[1], weight.shape
    n = b * h
    per_w = n // NW
    n_chunks = per_w // CHUNK
    mesh = plsc.VectorSubcoreMesh(core_axis_name="c", subcore_axis_name="s")

    @functools.partial(
        pl.kernel,
        mesh=mesh,
        compiler_params=pltpu.CompilerParams(use_tc_tiling_on_sc=False),
        out_type=jax.ShapeDtypeStruct((n, d), jnp.float32),
        scratch_types=[
            pltpu.VMEM((per_w,), jnp.int32),
            pltpu.VMEM((NBUF, CHUNK, d), jnp.float32),
            pltpu.SemaphoreType.DMA,
            pltpu.SemaphoreType.DMA,
        ],
    )
    def k(idx_hbm, tbl_hbm, out_hbm, idx_v, rows_v, gsem, ssem):
        wid = lax.axis_index("s") * NC + lax.axis_index("c")
        base = wid * per_w
        pltpu.sync_copy(idx_hbm.at[pl.ds(base, per_w)], idx_v)

        def gather_chunk(c, buf):
            for u in range(CHUNK // VG):
                vec = idx_v[pl.ds(c * CHUNK + u * VG, VG)]
                pltpu.async_copy(
                    tbl_hbm.at[vec], rows_v.at[buf, pl.ds(u * VG, VG)], gsem
                )

        for i in range(NBUF):
            gather_chunk(i, i)

        def body(j, carry):
            buf = lax.rem(j, NBUF)
            pltpu.make_async_copy(
                tbl_hbm.at[pl.ds(0, CHUNK)], rows_v.at[buf], gsem
            ).wait()
            pltpu.async_copy(
                rows_v.at[buf],
                out_hbm.at[pl.ds(base + j * CHUNK, CHUNK)],
                ssem,
            )

            @pl.when((j >= 1) & (j <= n_chunks - NBUF))
            def _():
                pltpu.make_async_copy(
                    rows_v.at[buf], out_hbm.at[pl.ds(base, CHUNK)], ssem
                ).wait()
                g = j - 1 + NBUF
                gather_chunk(g, lax.rem(g, NBUF))

            return carry

        lax.fori_loop(0, n_chunks, body, 0)

        for _ in range(NBUF):
            pltpu.make_async_copy(
                rows_v.at[0], out_hbm.at[pl.ds(base, CHUNK)], ssem
            ).wait()

    out2d = k(idx.reshape(n), weight)
    return out2d.reshape(b, h, d)


# speedup vs baseline: 1.0774x; 1.0037x over previous
import functools

import jax
import jax.numpy as jnp
from jax import lax
from jax.experimental import pallas as pl
from jax.experimental.pallas import tpu as pltpu
from jax.experimental.pallas import tpu_sc as plsc

NC = 2
NS = 16
NW = NC * NS
CHUNK = 128
VG = 16
NBUF = 12
SLACK = 3


def kernel(idx, weight):
    b, h, (v, d) = idx.shape[0], idx.shape[1], weight.shape
    n = b * h
    per_w = n // NW
    n_chunks = per_w // CHUNK
    mesh = plsc.VectorSubcoreMesh(core_axis_name="c", subcore_axis_name="s")

    @functools.partial(
        pl.kernel,
        mesh=mesh,
        compiler_params=pltpu.CompilerParams(use_tc_tiling_on_sc=False),
        out_type=jax.ShapeDtypeStruct((n, d), jnp.float32),
        scratch_types=[
            pltpu.VMEM((per_w,), jnp.int32),
            pltpu.VMEM((NBUF, CHUNK, d), jnp.float32),
            pltpu.SemaphoreType.DMA,
            pltpu.SemaphoreType.DMA,
        ],
    )
    def k(idx_hbm, tbl_hbm, out_hbm, idx_v, rows_v, gsem, ssem):
        wid = lax.axis_index("s") * NC + lax.axis_index("c")
        base = wid * per_w
        pltpu.sync_copy(idx_hbm.at[pl.ds(base, per_w)], idx_v)

        def gather_chunk(c, buf):
            for u in range(CHUNK // VG):
                vec = idx_v[pl.ds(c * CHUNK + u * VG, VG)]
                pltpu.async_copy(
                    tbl_hbm.at[vec], rows_v.at[buf, pl.ds(u * VG, VG)], gsem
                )

        for i in range(NBUF):
            gather_chunk(i, i)

        def body(j, carry):
            buf = lax.rem(j, NBUF)
            pltpu.make_async_copy(
                tbl_hbm.at[pl.ds(0, CHUNK)], rows_v.at[buf], gsem
            ).wait()
            pltpu.async_copy(
                rows_v.at[buf],
                out_hbm.at[pl.ds(base + j * CHUNK, CHUNK)],
                ssem,
            )

            @pl.when((j >= SLACK) & (j <= n_chunks - NBUF + SLACK - 1))
            def _():
                pltpu.make_async_copy(
                    rows_v.at[buf], out_hbm.at[pl.ds(base, CHUNK)], ssem
                ).wait()
                g = j - SLACK + NBUF
                gather_chunk(g, lax.rem(g, NBUF))

            return carry

        lax.fori_loop(0, n_chunks, body, 0)

        for _ in range(NBUF):
            pltpu.make_async_copy(
                rows_v.at[0], out_hbm.at[pl.ds(base, CHUNK)], ssem
            ).wait()

    out2d = k(idx.reshape(n), weight)
    return out2d.reshape(b, h, d)
